# Initial kernel scaffold; baseline (speedup 1.0000x reference)
import jax
import jax.numpy as jnp
from jax.experimental import pallas as pl


def kernel(node_features, edge_index, edge_attr, node_pos, W1, W2, ln_gamma, ln_beta):
    raise NotImplementedError("write your pallas kernel here")


# speedup vs baseline: 5.0512x; 5.0512x over previous
import functools

import jax
import jax.numpy as jnp
from jax import lax
from jax.experimental import pallas as pl
from jax.experimental.pallas import tpu as pltpu
from jax.experimental.pallas import tpu_sc as plsc

N_NODES = 10000
N_EDGES = 320000
D = 128
LN_EPS = 1e-5

NC, NS, L = 2, 16, 16
NW = NC * NS
EDGES_PER_TILE = N_EDGES // NW
CHUNK = 80
N_CHUNKS = EDGES_PER_TILE // CHUNK
ZROWS = 104
ROWS_BIG, ROWS_SMALL = 632, 624


def _sc_aggregate(x, src, dst, attr):
    mesh = plsc.VectorSubcoreMesh(core_axis_name="c", subcore_axis_name="s")

    @functools.partial(
        pl.kernel,
        out_type=jax.ShapeDtypeStruct((NC, N_NODES, D), jnp.float32),
        mesh=mesh,
        scratch_types=[
            pltpu.VMEM((CHUNK,), jnp.int32),
            pltpu.VMEM((CHUNK,), jnp.int32),
            pltpu.VMEM((EDGES_PER_TILE,), jnp.float32),
            pltpu.VMEM((CHUNK, D), jnp.float32),
            pltpu.VMEM((ZROWS, D), jnp.float32),
            pltpu.VMEM_SHARED((N_NODES, D), jnp.float32),
            pltpu.SemaphoreType.DMA,
        ],
    )
    def k(x_hbm, src_hbm, dst_hbm, attr_hbm, out_hbm,
          src_v, dst_v, attr_v, rows_v, z_v, acc_sh, sem):
        cid = lax.axis_index("c")
        sid = lax.axis_index("s")
        wid = sid * NC + cid
        tile_base = wid * EDGES_PER_TILE

        zero16 = jnp.zeros((L,), jnp.float32)

        def zbody(i, c):
            for j in range(D // L):
                z_v[i, pl.ds(j * L, L)] = zero16
            return c

        lax.fori_loop(0, ZROWS, zbody, 0)

        def zero_range(base, n):
            base = pl.multiple_of(base, 8)
            for r in range(n // ZROWS):
                pltpu.sync_copy(z_v, acc_sh.at[pl.ds(base + r * ZROWS, ZROWS)])
            rem = n - (n // ZROWS) * ZROWS
            if rem:
                pltpu.sync_copy(z_v.at[pl.ds(0, rem)],
                                acc_sh.at[pl.ds(base + (n // ZROWS) * ZROWS, rem)])

        @pl.when(sid < 2)
        def _():
            zero_range(sid * ROWS_BIG, ROWS_BIG)

        @pl.when(sid >= 2)
        def _():
            zero_range(ROWS_SMALL * sid + 16, ROWS_SMALL)

        pltpu.sync_copy(attr_hbm.at[pl.ds(tile_base, EDGES_PER_TILE)], attr_v)

        plsc.subcore_barrier()

        def chunk_body(ci, c):
            base = tile_base + ci * CHUNK
            pltpu.sync_copy(src_hbm.at[pl.ds(base, CHUNK)], src_v)
            pltpu.sync_copy(dst_hbm.at[pl.ds(base, CHUNK)], dst_v)
            pltpu.async_copy(x_hbm.at[src_v], rows_v, sem).wait()

            def ebody(g, c2):
                a16 = attr_v[pl.ds(ci * CHUNK + g * L, L)]
                for j in range(L):
                    a = jnp.full((L,), a16[j], jnp.float32)
                    row = g * L + j
                    for k in range(D // L):
                        rows_v[row, pl.ds(k * L, L)] = (
                            rows_v[row, pl.ds(k * L, L)] * a)
                return c2

            lax.fori_loop(0, CHUNK // L, ebody, 0)
            pltpu.sync_copy(rows_v, acc_sh.at[dst_v], add=True)
            return c

        lax.fori_loop(0, N_CHUNKS, chunk_body, 0)

        plsc.subcore_barrier()

        @pl.when(sid < 2)
        def _():
            base = pl.multiple_of(sid * ROWS_BIG, 8)
            pltpu.sync_copy(acc_sh.at[pl.ds(base, ROWS_BIG)],
                            out_hbm.at[cid].at[pl.ds(base, ROWS_BIG)])

        @pl.when(sid >= 2)
        def _():
            base = pl.multiple_of(ROWS_SMALL * sid + 16, 8)
            pltpu.sync_copy(acc_sh.at[pl.ds(base, ROWS_SMALL)],
                            out_hbm.at[cid].at[pl.ds(base, ROWS_SMALL)])

    return k(x, src, dst, attr)


def _tc_finish(parts, W1, W2, gamma, beta):
    def body(p_ref, w1_ref, w2_ref, g_ref, b_ref, o_ref):
        w = jnp.dot(w1_ref[...], w2_ref[...],
                    preferred_element_type=jnp.float32) * (1.0 / D)
        agg = p_ref[0] + p_ref[1]
        y = jnp.dot(agg, w, preferred_element_type=jnp.float32)
        mean = jnp.mean(y, axis=-1, keepdims=True)
        var = jnp.mean((y - mean) ** 2, axis=-1, keepdims=True)
        o_ref[...] = (y - mean) * lax.rsqrt(var + LN_EPS) * g_ref[...] + b_ref[...]

    return pl.pallas_call(
        body,
        out_shape=jax.ShapeDtypeStruct((N_NODES, D), jnp.float32),
    )(parts, W1, W2, gamma, beta)


def kernel(node_features, edge_index, edge_attr, node_pos, W1, W2, ln_gamma, ln_beta):
    src = edge_index[0]
    dst = edge_index[1]
    attr = edge_attr[:, 0]
    parts = _sc_aggregate(node_features, src, dst, attr)
    return _tc_finish(parts, W1, W2,
                      ln_gamma.reshape(1, D), ln_beta.reshape(1, D))


# speedup vs baseline: 10.5016x; 2.0790x over previous
import functools

import jax
import jax.numpy as jnp
from jax import lax
from jax.experimental import pallas as pl
from jax.experimental.pallas import tpu as pltpu
from jax.experimental.pallas import tpu_sc as plsc

N_NODES = 10000
N_EDGES = 320000
D = 128
LN_EPS = 1e-5

NC, NS, L = 2, 16, 16
NW = NC * NS
EDGES_PER_TILE = N_EDGES // NW
CHUNK = 80
N_CHUNKS = EDGES_PER_TILE // CHUNK
ROWS_BIG, ROWS_SMALL = 632, 624


def _sc_aggregate(x, src, dst, attr):
    mesh = plsc.VectorSubcoreMesh(core_axis_name="c", subcore_axis_name="s")

    @functools.partial(
        pl.kernel,
        out_type=jax.ShapeDtypeStruct((NC, N_NODES, D), jnp.float32),
        mesh=mesh,
        scratch_types=[
            pltpu.VMEM((CHUNK,), jnp.int32),
            pltpu.VMEM((CHUNK,), jnp.int32),
            pltpu.VMEM((CHUNK,), jnp.int32),
            pltpu.VMEM((CHUNK,), jnp.int32),
            pltpu.VMEM((CHUNK,), jnp.float32),
            pltpu.VMEM((CHUNK,), jnp.float32),
            pltpu.VMEM((CHUNK, D), jnp.float32),
            pltpu.VMEM((CHUNK, D), jnp.float32),
            pltpu.VMEM((CHUNK, D), jnp.float32),
            pltpu.VMEM((CHUNK, D), jnp.float32),
            pltpu.VMEM_SHARED((N_NODES, D), jnp.float32),
            pltpu.SemaphoreType.DMA,
            pltpu.SemaphoreType.DMA,
            pltpu.SemaphoreType.DMA,
            pltpu.SemaphoreType.DMA,
            pltpu.SemaphoreType.DMA,
            pltpu.SemaphoreType.DMA,
            pltpu.SemaphoreType.DMA,
            pltpu.SemaphoreType.DMA,
            pltpu.SemaphoreType.DMA,
            pltpu.SemaphoreType.DMA,
        ],
    )
    def k(x_hbm, src_hbm, dst_hbm, attr_hbm, out_hbm,
          sbuf0, sbuf1, dbuf0, dbuf1, abuf0, abuf1,
          in0, in1, out0, out1, acc_sh,
          gsem0, gsem1, ssem0, ssem1, srcsem0, srcsem1,
          dsem0, dsem1, asem0, asem1):
        cid = lax.axis_index("c")
        sid = lax.axis_index("s")
        wid = sid * NC + cid
        tile_base = wid * EDGES_PER_TILE

        pltpu.sync_copy(src_hbm.at[pl.ds(tile_base, CHUNK)], sbuf0)
        pltpu.sync_copy(src_hbm.at[pl.ds(tile_base + CHUNK, CHUNK)], sbuf1)
        pltpu.async_copy(x_hbm.at[sbuf0], in0, gsem0)
        pltpu.async_copy(x_hbm.at[sbuf1], in1, gsem1)
        pltpu.async_copy(attr_hbm.at[pl.ds(tile_base, CHUNK)], abuf0, asem0)
        pltpu.async_copy(attr_hbm.at[pl.ds(tile_base + CHUNK, CHUNK)],
                         abuf1, asem1)

        zero16 = jnp.zeros((L,), jnp.float32)

        def zbody(i, c):
            for j in range(D // L):
                out0[i, pl.ds(j * L, L)] = zero16
            return c

        lax.fori_loop(0, CHUNK, zbody, 0)

        def zero_range(base, n):
            base = pl.multiple_of(base, 8)
            for r in range(n // CHUNK):
                pltpu.sync_copy(out0, acc_sh.at[pl.ds(base + r * CHUNK, CHUNK)])
            rem = n - (n // CHUNK) * CHUNK
            if rem:
                pltpu.sync_copy(out0.at[pl.ds(0, rem)],
                                acc_sh.at[pl.ds(base + (n // CHUNK) * CHUNK, rem)])

        @pl.when(sid < 2)
        def _():
            zero_range(sid * ROWS_BIG, ROWS_BIG)

        @pl.when(sid >= 2)
        def _():
            zero_range(ROWS_SMALL * sid + 16, ROWS_SMALL)

        plsc.subcore_barrier()

        def handle(c, sbuf, dbuf, abuf, inb, outb,
                   gsem, ssem, srcsem, dsem, asem):
            pltpu.make_async_copy(x_hbm.at[sbuf], inb, gsem).wait()

            @pl.when(c >= 2)
            def _():
                pltpu.make_async_copy(outb, acc_sh.at[dbuf], ssem).wait()

            pltpu.async_copy(
                dst_hbm.at[pl.ds(tile_base + c * CHUNK, CHUNK)], dbuf, dsem)

            @pl.when(c + 2 < N_CHUNKS)
            def _():
                pltpu.async_copy(
                    src_hbm.at[pl.ds(tile_base + (c + 2) * CHUNK, CHUNK)],
                    sbuf, srcsem)

            pltpu.make_async_copy(
                attr_hbm.at[pl.ds(tile_base, CHUNK)], abuf, asem).wait()

            def gbody(g, cc):
                a16 = abuf[pl.ds(g * L, L)]
                for j in range(L):
                    a = jnp.full((L,), a16[j], jnp.float32)
                    row = g * L + j
                    for kk in range(D // L):
                        outb[row, pl.ds(kk * L, L)] = (
                            inb[row, pl.ds(kk * L, L)] * a)
                return cc

            lax.fori_loop(0, CHUNK // L, gbody, 0)

            @pl.when(c + 2 < N_CHUNKS)
            def _():
                pltpu.async_copy(
                    attr_hbm.at[pl.ds(tile_base + (c + 2) * CHUNK, CHUNK)],
                    abuf, asem)
                pltpu.make_async_copy(
                    src_hbm.at[pl.ds(tile_base, CHUNK)], sbuf, srcsem).wait()
                pltpu.async_copy(x_hbm.at[sbuf], inb, gsem)

            pltpu.make_async_copy(
                dst_hbm.at[pl.ds(tile_base, CHUNK)], dbuf, dsem).wait()
            pltpu.async_copy(outb, acc_sh.at[dbuf], ssem, add=True)

        def pair(p, c):
            handle(2 * p, sbuf0, dbuf0, abuf0, in0, out0,
                   gsem0, ssem0, srcsem0, dsem0, asem0)
            handle(2 * p + 1, sbuf1, dbuf1, abuf1, in1, out1,
                   gsem1, ssem1, srcsem1, dsem1, asem1)
            return c

        lax.fori_loop(0, (N_CHUNKS - 1) // 2, pair, 0)
        handle(jnp.int32(N_CHUNKS - 1), sbuf0, dbuf0, abuf0, in0, out0,
               gsem0, ssem0, srcsem0, dsem0, asem0)

        pltpu.make_async_copy(out1, acc_sh.at[dbuf1], ssem1).wait()
        pltpu.make_async_copy(out0, acc_sh.at[dbuf0], ssem0).wait()

        plsc.subcore_barrier()

        @pl.when(sid < 2)
        def _():
            base = pl.multiple_of(sid * ROWS_BIG, 8)
            pltpu.sync_copy(acc_sh.at[pl.ds(base, ROWS_BIG)],
                            out_hbm.at[cid].at[pl.ds(base, ROWS_BIG)])

        @pl.when(sid >= 2)
        def _():
            base = pl.multiple_of(ROWS_SMALL * sid + 16, 8)
            pltpu.sync_copy(acc_sh.at[pl.ds(base, ROWS_SMALL)],
                            out_hbm.at[cid].at[pl.ds(base, ROWS_SMALL)])

    return k(x, src, dst, attr)


def _tc_finish(parts, W1, W2, gamma, beta):
    def body(p_ref, w1_ref, w2_ref, g_ref, b_ref, o_ref):
        w = jnp.dot(w1_ref[...], w2_ref[...],
                    preferred_element_type=jnp.float32) * (1.0 / D)
        agg = p_ref[0] + p_ref[1]
        y = jnp.dot(agg, w, preferred_element_type=jnp.float32)
        mean = jnp.mean(y, axis=-1, keepdims=True)
        var = jnp.mean((y - mean) ** 2, axis=-1, keepdims=True)
        o_ref[...] = (y - mean) * lax.rsqrt(var + LN_EPS) * g_ref[...] + b_ref[...]

    return pl.pallas_call(
        body,
        out_shape=jax.ShapeDtypeStruct((N_NODES, D), jnp.float32),
    )(parts, W1, W2, gamma, beta)


def kernel(node_features, edge_index, edge_attr, node_pos, W1, W2, ln_gamma, ln_beta):
    src = edge_index[0]
    dst = edge_index[1]
    attr = edge_attr[:, 0]
    parts = _sc_aggregate(node_features, src, dst, attr)
    return _tc_finish(parts, W1, W2,
                      ln_gamma.reshape(1, D), ln_beta.reshape(1, D))


# speedup vs baseline: 12.1101x; 1.1532x over previous
import functools

import jax
import jax.numpy as jnp
from jax import lax
from jax.experimental import pallas as pl
from jax.experimental.pallas import tpu as pltpu
from jax.experimental.pallas import tpu_sc as plsc

N_NODES = 10000
N_EDGES = 320000
D = 128
LN_EPS = 1e-5

NC, NS, L = 2, 16, 16
NW = NC * NS
EDGES_PER_TILE = N_EDGES // NW
CHUNK = 80
N_CHUNKS = EDGES_PER_TILE // CHUNK
ROWS_BIG, ROWS_SMALL = 632, 624


def _sc_aggregate(x, ei, attr):
    mesh = plsc.VectorSubcoreMesh(core_axis_name="c", subcore_axis_name="s")

    @functools.partial(
        pl.kernel,
        out_type=jax.ShapeDtypeStruct((NC, N_NODES, D), jnp.float32),
        mesh=mesh,
        scratch_types=[
            pltpu.VMEM((CHUNK,), jnp.int32),
            pltpu.VMEM((CHUNK,), jnp.int32),
            pltpu.VMEM((CHUNK,), jnp.int32),
            pltpu.VMEM((CHUNK,), jnp.int32),
            pltpu.VMEM((CHUNK,), jnp.float32),
            pltpu.VMEM((CHUNK,), jnp.float32),
            pltpu.VMEM((CHUNK, D), jnp.float32),
            pltpu.VMEM((CHUNK, D), jnp.float32),
            pltpu.VMEM((CHUNK, D), jnp.float32),
            pltpu.VMEM((CHUNK, D), jnp.float32),
            pltpu.VMEM_SHARED((N_NODES, D), jnp.float32),
            pltpu.SemaphoreType.DMA,
            pltpu.SemaphoreType.DMA,
            pltpu.SemaphoreType.DMA,
            pltpu.SemaphoreType.DMA,
            pltpu.SemaphoreType.DMA,
            pltpu.SemaphoreType.DMA,
            pltpu.SemaphoreType.DMA,
            pltpu.SemaphoreType.DMA,
            pltpu.SemaphoreType.DMA,
            pltpu.SemaphoreType.DMA,
        ],
    )
    def k(x_hbm, ei_hbm, attr_hbm, out_hbm,
          sbuf0, sbuf1, dbuf0, dbuf1, abuf0, abuf1,
          in0, in1, out0, out1, acc_sh,
          gsem0, gsem1, ssem0, ssem1, srcsem0, srcsem1,
          dsem0, dsem1, asem0, asem1):
        cid = lax.axis_index("c")
        sid = lax.axis_index("s")
        wid = sid * NC + cid
        tile_base = wid * EDGES_PER_TILE

        pltpu.sync_copy(ei_hbm.at[pl.ds(tile_base, CHUNK)], sbuf0)
        pltpu.sync_copy(ei_hbm.at[pl.ds(tile_base + CHUNK, CHUNK)], sbuf1)
        pltpu.async_copy(x_hbm.at[sbuf0], in0, gsem0)
        pltpu.async_copy(x_hbm.at[sbuf1], in1, gsem1)
        pltpu.async_copy(attr_hbm.at[pl.ds(tile_base, CHUNK)], abuf0, asem0)
        pltpu.async_copy(attr_hbm.at[pl.ds(tile_base + CHUNK, CHUNK)],
                         abuf1, asem1)

        zero16 = jnp.zeros((L,), jnp.float32)

        def zbody(i, c):
            for j in range(D // L):
                out0[i, pl.ds(j * L, L)] = zero16
            return c

        lax.fori_loop(0, CHUNK, zbody, 0)

        def zero_range(base, n):
            base = pl.multiple_of(base, 8)
            for r in range(n // CHUNK):
                pltpu.sync_copy(out0, acc_sh.at[pl.ds(base + r * CHUNK, CHUNK)])
            rem = n - (n // CHUNK) * CHUNK
            if rem:
                pltpu.sync_copy(out0.at[pl.ds(0, rem)],
                                acc_sh.at[pl.ds(base + (n // CHUNK) * CHUNK, rem)])

        @pl.when(sid < 2)
        def _():
            zero_range(sid * ROWS_BIG, ROWS_BIG)

        @pl.when(sid >= 2)
        def _():
            zero_range(ROWS_SMALL * sid + 16, ROWS_SMALL)

        plsc.subcore_barrier()

        def handle(c, sbuf, dbuf, abuf, inb, outb,
                   gsem, ssem, srcsem, dsem, asem):
            pltpu.make_async_copy(x_hbm.at[sbuf], inb, gsem).wait()

            @pl.when(c >= 2)
            def _():
                pltpu.make_async_copy(outb, acc_sh.at[dbuf], ssem).wait()

            pltpu.async_copy(
                ei_hbm.at[pl.ds(N_EDGES + tile_base + c * CHUNK, CHUNK)], dbuf, dsem)

            @pl.when(c + 2 < N_CHUNKS)
            def _():
                pltpu.async_copy(
                    ei_hbm.at[pl.ds(tile_base + (c + 2) * CHUNK, CHUNK)],
                    sbuf, srcsem)

            pltpu.make_async_copy(
                attr_hbm.at[pl.ds(tile_base, CHUNK)], abuf, asem).wait()

            for g in range(CHUNK // L):
                a16 = abuf[pl.ds(g * L, L)]
                for j in range(L):
                    a = a16.at[jnp.full((L,), j, jnp.int32)].get(
                        mode="promise_in_bounds")
                    row = g * L + j
                    for kk in range(D // L):
                        outb[row, pl.ds(kk * L, L)] = (
                            inb[row, pl.ds(kk * L, L)] * a)

            @pl.when(c + 2 < N_CHUNKS)
            def _():
                pltpu.async_copy(
                    attr_hbm.at[pl.ds(tile_base + (c + 2) * CHUNK, CHUNK)],
                    abuf, asem)
                pltpu.make_async_copy(
                    ei_hbm.at[pl.ds(tile_base, CHUNK)], sbuf, srcsem).wait()
                pltpu.async_copy(x_hbm.at[sbuf], inb, gsem)

            pltpu.make_async_copy(
                ei_hbm.at[pl.ds(N_EDGES + tile_base, CHUNK)], dbuf, dsem).wait()
            pltpu.async_copy(outb, acc_sh.at[dbuf], ssem, add=True)

        def pair(p, c):
            handle(2 * p, sbuf0, dbuf0, abuf0, in0, out0,
                   gsem0, ssem0, srcsem0, dsem0, asem0)
            handle(2 * p + 1, sbuf1, dbuf1, abuf1, in1, out1,
                   gsem1, ssem1, srcsem1, dsem1, asem1)
            return c

        lax.fori_loop(0, (N_CHUNKS - 1) // 2, pair, 0)
        handle(jnp.int32(N_CHUNKS - 1), sbuf0, dbuf0, abuf0, in0, out0,
               gsem0, ssem0, srcsem0, dsem0, asem0)

        pltpu.make_async_copy(out1, acc_sh.at[dbuf1], ssem1).wait()
        pltpu.make_async_copy(out0, acc_sh.at[dbuf0], ssem0).wait()

        plsc.subcore_barrier()

        @pl.when(sid < 2)
        def _():
            base = pl.multiple_of(sid * ROWS_BIG, 8)
            pltpu.sync_copy(acc_sh.at[pl.ds(base, ROWS_BIG)],
                            out_hbm.at[cid].at[pl.ds(base, ROWS_BIG)])

        @pl.when(sid >= 2)
        def _():
            base = pl.multiple_of(ROWS_SMALL * sid + 16, 8)
            pltpu.sync_copy(acc_sh.at[pl.ds(base, ROWS_SMALL)],
                            out_hbm.at[cid].at[pl.ds(base, ROWS_SMALL)])

    return k(x, ei, attr)


def _tc_finish(parts, W1, W2, gamma, beta):
    def body(p_ref, w1_ref, w2_ref, g_ref, b_ref, o_ref):
        w = jnp.dot(w1_ref[...], w2_ref[...],
                    preferred_element_type=jnp.float32) * (1.0 / D)
        agg = p_ref[0] + p_ref[1]
        y = jnp.dot(agg, w, preferred_element_type=jnp.float32)
        mean = jnp.mean(y, axis=-1, keepdims=True)
        var = jnp.mean((y - mean) ** 2, axis=-1, keepdims=True)
        o_ref[...] = (y - mean) * lax.rsqrt(var + LN_EPS) * g_ref[...] + b_ref[...]

    return pl.pallas_call(
        body,
        out_shape=jax.ShapeDtypeStruct((N_NODES, D), jnp.float32),
    )(parts, W1, W2, gamma, beta)


def kernel(node_features, edge_index, edge_attr, node_pos, W1, W2, ln_gamma, ln_beta):
    ei = edge_index.reshape(2 * N_EDGES)
    attr = edge_attr.reshape(N_EDGES)
    parts = _sc_aggregate(node_features, ei, attr)
    return _tc_finish(parts, W1, W2,
                      ln_gamma.reshape(1, D), ln_beta.reshape(1, D))
